# baseline (speedup 1.0000x reference)
import functools

import jax
import jax.numpy as jnp
from jax.experimental import pallas as pl


def _energy_kernel(n_elements, ft_ref, mass_ref, be_ref, ke_ref, ie_ref):
    vx = ft_ref[3:4, :]
    vy = ft_ref[4:5, :]
    vz = ft_ref[5:6, :]
    mass = mass_ref[...]
    sq = vx * vx + vy * vy + vz * vz
    ke_ref[...] = 0.5 * jnp.sum(mass * sq, keepdims=True)
    ie_ref[...] = jnp.float32(n_elements) * be_ref[...]


def kernel(x, node_mass, element_to_nodes, element_materials,
           W1n, b1n, W2n, b2n, gn, bn, W1e, b1e, W2e, b2e, ge, be):
    e = element_to_nodes.shape[0]
    feat_t = x[:, :, -1].T
    mass_t = node_mass.T
    ke, ie = pl.pallas_call(
        functools.partial(_energy_kernel, e),
        out_shape=(
            jax.ShapeDtypeStruct((1, 1), jnp.float32),
            jax.ShapeDtypeStruct((1, 1), jnp.float32),
        ),
    )(feat_t, mass_t, be.reshape(1, 1))
    return (ke[0, 0], ie[0, 0])
